# baseline (speedup 1.0000x reference)
import functools

import jax
import jax.numpy as jnp
from jax import lax
from jax.experimental import pallas as pl
from jax.experimental.pallas import tpu as pltpu
from jax.experimental.pallas import tpu_sc as plsc

N = 10000
E = 320000
H = 128
R = 200
L = 2
SLOPE_NEG = (1.0 / 8.0 + 1.0 / 3.0) / 2.0

NC = 2
NS = 16
NW = NC * NS
CHUNK = 128
EP = NW * CHUNK * 80
NCH = EP // CHUNK
CPW = NCH // NW
ROWS_PER_TILE = 632
NP = NS * ROWS_PER_TILE
NR = N * R
CW_PER_TILE = NR // NS


def _gru_body(nei, wu, uu, bu, wr, ur, br, wh, uh, bh, w_out):
    q = nei[0]
    upd = jax.nn.sigmoid(jnp.dot(wu[0] + uu[0], q, preferred_element_type=jnp.float32) + bu[0])
    rst = jax.nn.sigmoid(jnp.dot(wr[0] + ur[0], q, preferred_element_type=jnp.float32) + br[0])
    hcap = jnp.tanh(
        jnp.dot(wh[0], q, preferred_element_type=jnp.float32)
        + jnp.dot(uh[0], rst * q, preferred_element_type=jnp.float32)
        + bh[0]
    )
    w_out[0] = (1.0 - upd) * q + upd * hcap


def _gru(nei_W, Wu, Uu, bu, Wr, Ur, br, Wh, Uh, bh):
    mat_spec = pl.BlockSpec((1, H, H), lambda i: (i, 0, 0))
    return pl.pallas_call(
        _gru_body,
        grid=(L,),
        in_specs=[mat_spec] * 10,
        out_specs=mat_spec,
        out_shape=jax.ShapeDtypeStruct((L, H, H), jnp.float32),
    )(nei_W, Wu, Uu, bu, Wr, Ur, br, Wh, Uh, bh)


LANES = 16
CPC = NCH // NC
VECS = CHUNK // LANES
CWO = 125056


def _sc_count_body(aux_hbm, out_hbm, idx_v, cnt_v, sems):
    c = lax.axis_index("c")
    s = lax.axis_index("s")
    base = c * CPC
    lo = s * CW_PER_TILE
    obase = (c * NS + s) * CWO
    (sem_i,) = sems
    ones16 = jnp.full((LANES,), 1.0, jnp.float32)

    def zstep(i, carry):
        cnt_v[pl.ds(i * LANES, LANES)] = jnp.zeros((LANES,), jnp.float32)
        return carry

    lax.fori_loop(0, CWO // LANES, zstep, 0)

    def start_idx(buf, row):
        pltpu.async_copy(aux_hbm.at[row], idx_v.at[buf], sem_i[buf])

    def wait_idx(buf, row):
        pltpu.make_async_copy(aux_hbm.at[row], idx_v.at[buf], sem_i[buf]).wait()

    start_idx(0, base)

    def step(j, carry):
        for b in range(2):
            ch = 2 * j + b
            nb = 1 - b

            @pl.when(ch + 1 < CPC)
            def _():
                start_idx(nb, base + ch + 1)

            wait_idx(b, base + ch)
            for g in range(VECS):
                cidx = idx_v[b, 3, pl.ds(g * LANES, LANES)]
                local = cidx - lo
                mask = (local >= 0) & (local < CW_PER_TILE)
                local = jnp.where(mask, local, CW_PER_TILE)
                plsc.addupdate_scatter(cnt_v, [local], ones16)
        return carry

    lax.fori_loop(0, CPC // 2, step, 0)
    pltpu.sync_copy(cnt_v, out_hbm.at[pl.ds(obase, CWO)])


@functools.partial(
    pl.kernel,
    out_type=jax.ShapeDtypeStruct((NC * NS * CWO,), jnp.float32),
    mesh=plsc.VectorSubcoreMesh(core_axis_name="c", subcore_axis_name="s"),
    scratch_types=[
        pltpu.VMEM((2, 4, CHUNK), jnp.int32),
        pltpu.VMEM((CWO,), jnp.float32),
        [[pltpu.SemaphoreType.DMA] * 2],
    ],
    compiler_params=pltpu.CompilerParams(needs_layout_passes=False),
)
def _sc_count(aux_hbm, out_hbm, idx_v, cnt_v, sems):
    _sc_count_body(aux_hbm, out_hbm, idx_v, cnt_v, sems)


def _sc_accum_body(h_hbm, aux_hbm, zeros_hbm, out_hbm, idx_v, hrows, acc_sh, sems):
    c = lax.axis_index("c")
    s = lax.axis_index("s")
    w = s * NC + c
    base = w * CPW
    sem_h, sem_s = sems

    def gathers(buf, row):
        pltpu.sync_copy(aux_hbm.at[row], idx_v.at[buf])
        pltpu.async_copy(h_hbm.at[idx_v.at[buf, 0]], hrows.at[buf], sem_h[buf])

    def wait_gather(buf):
        pltpu.make_async_copy(h_hbm.at[idx_v.at[buf, 0]], hrows.at[buf], sem_h[buf]).wait()

    def wait_scatter(buf):
        pltpu.make_async_copy(hrows.at[buf], acc_sh.at[idx_v.at[buf, 2]], sem_s[buf]).wait()

    pltpu.sync_copy(zeros_hbm.at[pl.ds(s * ROWS_PER_TILE, ROWS_PER_TILE)],
                    acc_sh.at[pl.ds(s * ROWS_PER_TILE, ROWS_PER_TILE)])
    plsc.subcore_barrier()

    gathers(0, base)

    def step(j, carry):
        for b in range(2):
            ch = 2 * j + b
            nb = 1 - b
            @pl.when(ch + 1 < CPW)
            def _():
                @pl.when(ch >= 1)
                def _():
                    wait_scatter(nb)
                gathers(nb, base + ch + 1)
            wait_gather(b)
            pltpu.async_copy(hrows.at[b], acc_sh.at[idx_v.at[b, 2]], sem_s[b], add=True)
        return carry

    lax.fori_loop(0, CPW // 2, step, 0)
    wait_scatter(0)
    wait_scatter(1)
    plsc.subcore_barrier()
    pltpu.sync_copy(acc_sh.at[pl.ds(s * ROWS_PER_TILE, ROWS_PER_TILE)],
                    out_hbm.at[c, pl.ds(s * ROWS_PER_TILE, ROWS_PER_TILE)])


@functools.partial(
    pl.kernel,
    out_type=jax.ShapeDtypeStruct((NC, NP, H), jnp.float32),
    mesh=plsc.VectorSubcoreMesh(core_axis_name="c", subcore_axis_name="s"),
    scratch_types=[
        pltpu.VMEM((2, 4, CHUNK), jnp.int32),
        pltpu.VMEM((2, CHUNK, H), jnp.float32),
        pltpu.VMEM_SHARED((NP, H), jnp.float32),
        [[pltpu.SemaphoreType.DMA] * 2] * 2,
    ],
)
def _sc_accum(h_hbm, aux_hbm, zeros_hbm, out_hbm, idx_v, hrows, acc_sh, sems):
    _sc_accum_body(h_hbm, aux_hbm, zeros_hbm, out_hbm, idx_v, hrows, acc_sh, sems)


BN = 1000


def _combine_body(p_ref, c_ref, rel_ref, w_ref, o_ref):
    acc = p_ref[0] + p_ref[1]
    cnt = c_ref[0] + c_ref[1]
    acc = acc - jnp.dot(cnt, rel_ref[...], preferred_element_type=jnp.float32)
    o = jnp.dot(acc, w_ref[...], preferred_element_type=jnp.float32)
    o_ref[...] = jnp.where(o >= 0, o, o * SLOPE_NEG)


def _combine(p, cnt, rel, w):
    return pl.pallas_call(
        _combine_body,
        grid=(N // BN,),
        in_specs=[
            pl.BlockSpec((2, BN, H), lambda i: (0, i, 0)),
            pl.BlockSpec((2, BN, R), lambda i: (0, i, 0)),
            pl.BlockSpec((R, H), lambda i: (0, 0)),
            pl.BlockSpec((H, H), lambda i: (0, 0)),
        ],
        out_specs=pl.BlockSpec((BN, H), lambda i: (i, 0)),
        out_shape=jax.ShapeDtypeStruct((N, H), jnp.float32),
    )(p, cnt, rel, w)


def kernel(init_ent_emb, init_rel_emb, edge_index, edge_type, node_id,
           Wu, Uu, bu, Wr, Ur, br, Wh, Uh, bh, nei_W):
    h = jnp.take(init_ent_emb, node_id, axis=0)
    w_ev = _gru(nei_W, Wu, Uu, bu, Wr, Ur, br, Wh, Uh, bh)
    pad = EP - E
    src = jnp.concatenate([edge_index[0], jnp.zeros((pad,), jnp.int32)])
    dst = jnp.concatenate([edge_index[1], jnp.full((pad,), N, jnp.int32)])
    et = jnp.concatenate([edge_type, jnp.zeros((pad,), jnp.int32)])
    cidx = dst * R + et
    aux = jnp.stack([src.reshape(NCH, CHUNK), et.reshape(NCH, CHUNK),
                     dst.reshape(NCH, CHUNK), cidx.reshape(NCH, CHUNK)],
                    axis=1)
    zeros_a = jnp.zeros((NP, H), jnp.float32)
    cnt = _sc_count(aux)
    cnt_m = cnt.reshape(NC, NS, CWO)[:, :, :CW_PER_TILE].reshape(NC, N, R)
    for l in range(L):
        p = _sc_accum(h, aux, zeros_a)
        h = _combine(p[:, :N], cnt_m, init_rel_emb[l], w_ev[l])
    return h


# speedup vs baseline: 1.8890x; 1.8890x over previous
import functools

import jax
import jax.numpy as jnp
from jax import lax
from jax.experimental import pallas as pl
from jax.experimental.pallas import tpu as pltpu
from jax.experimental.pallas import tpu_sc as plsc

N = 10000
E = 320000
H = 128
R = 200
L = 2
SLOPE_NEG = (1.0 / 8.0 + 1.0 / 3.0) / 2.0

NC = 2
NS = 16
NW = NC * NS
LANES = 16
CHUNK = 112
CPW = 90
NCH = NW * CPW
EP = NCH * CHUNK
CPC = NCH // NC
VECS = CHUNK // LANES
ROWS_PER_TILE = 632
NP = NS * ROWS_PER_TILE
NR = N * R
CW_PER_TILE = NR // NS
CWO = 125056
CG = 16
NF = CPC // CG


def _gru_body(nei, wu, uu, bu, wr, ur, br, wh, uh, bh, w_out):
    q = nei[0]
    upd = jax.nn.sigmoid(jnp.dot(wu[0] + uu[0], q, preferred_element_type=jnp.float32) + bu[0])
    rst = jax.nn.sigmoid(jnp.dot(wr[0] + ur[0], q, preferred_element_type=jnp.float32) + br[0])
    hcap = jnp.tanh(
        jnp.dot(wh[0], q, preferred_element_type=jnp.float32)
        + jnp.dot(uh[0], rst * q, preferred_element_type=jnp.float32)
        + bh[0]
    )
    w_out[0] = (1.0 - upd) * q + upd * hcap


def _gru(nei_W, Wu, Uu, bu, Wr, Ur, br, Wh, Uh, bh):
    mat_spec = pl.BlockSpec((1, H, H), lambda i: (i, 0, 0))
    return pl.pallas_call(
        _gru_body,
        grid=(L,),
        in_specs=[mat_spec] * 10,
        out_specs=mat_spec,
        out_shape=jax.ShapeDtypeStruct((L, H, H), jnp.float32),
    )(nei_W, Wu, Uu, bu, Wr, Ur, br, Wh, Uh, bh)


def _sc_count_body(cidx_hbm, out_hbm, idx_v, cnt_v, sems):
    c = lax.axis_index("c")
    s = lax.axis_index("s")
    fbase = c * NF
    lo = s * CW_PER_TILE
    obase = (c * NS + s) * CWO
    (sem_i,) = sems
    ones16 = jnp.full((LANES,), 1.0, jnp.float32)

    def zstep(i, carry):
        cnt_v[pl.ds(i * LANES, LANES)] = jnp.zeros((LANES,), jnp.float32)
        return carry

    lax.fori_loop(0, CWO // LANES, zstep, 0)

    def start_idx(buf, f):
        pltpu.async_copy(cidx_hbm.at[pl.ds((fbase + f) * CG, CG)], idx_v.at[buf],
                         sem_i[buf])

    def wait_idx(buf, f):
        pltpu.make_async_copy(cidx_hbm.at[pl.ds((fbase + f) * CG, CG)],
                              idx_v.at[buf], sem_i[buf]).wait()

    start_idx(0, 0)

    def step(j, carry):
        for b in range(2):
            f = 2 * j + b
            nb = 1 - b

            @pl.when(f + 1 < NF)
            def _():
                start_idx(nb, f + 1)

            wait_idx(b, f)
            for q in range(CG):
                for g in range(VECS):
                    cidx = idx_v[b, q, pl.ds(g * LANES, LANES)]
                    local = cidx - lo
                    mask = (local >= 0) & (local < CW_PER_TILE)
                    local = jnp.where(mask, local, CW_PER_TILE)
                    plsc.addupdate_scatter(cnt_v, [local], ones16)
        return carry

    lax.fori_loop(0, NF // 2, step, 0)
    pltpu.sync_copy(cnt_v, out_hbm.at[pl.ds(obase, CWO)])


@functools.partial(
    pl.kernel,
    out_type=jax.ShapeDtypeStruct((NC * NS * CWO,), jnp.float32),
    mesh=plsc.VectorSubcoreMesh(core_axis_name="c", subcore_axis_name="s"),
    scratch_types=[
        pltpu.VMEM((2, CG, CHUNK), jnp.int32),
        pltpu.VMEM((CWO,), jnp.float32),
        [[pltpu.SemaphoreType.DMA] * 2],
    ],
    compiler_params=pltpu.CompilerParams(needs_layout_passes=False),
)
def _sc_count(cidx_hbm, out_hbm, idx_v, cnt_v, sems):
    _sc_count_body(cidx_hbm, out_hbm, idx_v, cnt_v, sems)


def _sc_accum_body(h_hbm, aux_hbm, zeros_hbm, out_hbm, idx_v, hrows, acc_sh, sems):
    c = lax.axis_index("c")
    s = lax.axis_index("s")
    w = s * NC + c
    base = w * CPW
    sem_i, sem_h, sem_s = sems

    def wait_scatter(buf):
        pltpu.make_async_copy(hrows.at[buf], acc_sh.at[idx_v.at[buf, 2]],
                              sem_s[buf]).wait()

    def prep(c2, buf):
        @pl.when(c2 < CPW)
        def _():
            @pl.when(c2 >= 3)
            def _():
                wait_scatter(buf)
            pltpu.async_copy(aux_hbm.at[base + c2], idx_v.at[buf], sem_i[buf])

    def gath(c1, buf):
        @pl.when(c1 < CPW)
        def _():
            pltpu.make_async_copy(aux_hbm.at[base + c1], idx_v.at[buf],
                                  sem_i[buf]).wait()
            pltpu.async_copy(h_hbm.at[idx_v.at[buf, 0]], hrows.at[buf], sem_h[buf])

    pltpu.sync_copy(zeros_hbm.at[pl.ds(s * ROWS_PER_TILE, ROWS_PER_TILE)],
                    acc_sh.at[pl.ds(s * ROWS_PER_TILE, ROWS_PER_TILE)])
    plsc.subcore_barrier()

    prep(0, 0)
    prep(1, 1)
    gath(0, 0)

    def step(j, carry):
        for b in range(3):
            ch = 3 * j + b
            prep(ch + 2, (b + 2) % 3)
            gath(ch + 1, (b + 1) % 3)
            pltpu.make_async_copy(h_hbm.at[idx_v.at[b, 0]], hrows.at[b],
                                  sem_h[b]).wait()
            pltpu.async_copy(hrows.at[b], acc_sh.at[idx_v.at[b, 2]], sem_s[b],
                             add=True)
        return carry

    lax.fori_loop(0, CPW // 3, step, 0)
    for ch in (CPW - 3, CPW - 2, CPW - 1):
        wait_scatter(ch % 3)
    plsc.subcore_barrier()
    pltpu.sync_copy(acc_sh.at[pl.ds(s * ROWS_PER_TILE, ROWS_PER_TILE)],
                    out_hbm.at[c, pl.ds(s * ROWS_PER_TILE, ROWS_PER_TILE)])


@functools.partial(
    pl.kernel,
    out_type=jax.ShapeDtypeStruct((NC, NP, H), jnp.float32),
    mesh=plsc.VectorSubcoreMesh(core_axis_name="c", subcore_axis_name="s"),
    scratch_types=[
        pltpu.VMEM((3, 4, CHUNK), jnp.int32),
        pltpu.VMEM((3, CHUNK, H), jnp.float32),
        pltpu.VMEM_SHARED((NP, H), jnp.float32),
        [[pltpu.SemaphoreType.DMA] * 3] * 3,
    ],
)
def _sc_accum(h_hbm, aux_hbm, zeros_hbm, out_hbm, idx_v, hrows, acc_sh, sems):
    _sc_accum_body(h_hbm, aux_hbm, zeros_hbm, out_hbm, idx_v, hrows, acc_sh, sems)


BN = 1000


def _combine_body(p_ref, c_ref, rel_ref, w_ref, o_ref):
    acc = p_ref[0] + p_ref[1]
    cnt = c_ref[0] + c_ref[1]
    acc = acc - jnp.dot(cnt, rel_ref[...], preferred_element_type=jnp.float32)
    o = jnp.dot(acc, w_ref[...], preferred_element_type=jnp.float32)
    o_ref[...] = jnp.where(o >= 0, o, o * SLOPE_NEG)


def _combine(p, cnt, rel, w):
    return pl.pallas_call(
        _combine_body,
        grid=(N // BN,),
        in_specs=[
            pl.BlockSpec((2, BN, H), lambda i: (0, i, 0)),
            pl.BlockSpec((2, BN, R), lambda i: (0, i, 0)),
            pl.BlockSpec((R, H), lambda i: (0, 0)),
            pl.BlockSpec((H, H), lambda i: (0, 0)),
        ],
        out_specs=pl.BlockSpec((BN, H), lambda i: (i, 0)),
        out_shape=jax.ShapeDtypeStruct((N, H), jnp.float32),
    )(p, cnt, rel, w)


def kernel(init_ent_emb, init_rel_emb, edge_index, edge_type, node_id,
           Wu, Uu, bu, Wr, Ur, br, Wh, Uh, bh, nei_W):
    h = jnp.take(init_ent_emb, node_id, axis=0)
    w_ev = _gru(nei_W, Wu, Uu, bu, Wr, Ur, br, Wh, Uh, bh)
    pad = EP - E
    src = jnp.concatenate([edge_index[0], jnp.zeros((pad,), jnp.int32)])
    dst = jnp.concatenate([edge_index[1], jnp.full((pad,), N, jnp.int32)])
    et = jnp.concatenate([edge_type, jnp.zeros((pad,), jnp.int32)])
    cidx = dst * R + et
    aux = jnp.stack([src.reshape(NCH, CHUNK), et.reshape(NCH, CHUNK),
                     dst.reshape(NCH, CHUNK), cidx.reshape(NCH, CHUNK)],
                    axis=1)
    zeros_a = jnp.zeros((NP, H), jnp.float32)
    cnt = _sc_count(cidx.reshape(NCH, CHUNK))
    cnt_m = cnt.reshape(NC, NS, CWO)[:, :, :CW_PER_TILE].reshape(NC, N, R)
    for l in range(L):
        p = _sc_accum(h, aux, zeros_a)
        h = _combine(p[:, :N], cnt_m, init_rel_emb[l], w_ev[l])
    return h


# speedup vs baseline: 1.9908x; 1.0539x over previous
import functools

import jax
import jax.numpy as jnp
from jax import lax
from jax.experimental import pallas as pl
from jax.experimental.pallas import tpu as pltpu
from jax.experimental.pallas import tpu_sc as plsc

N = 10000
E = 320000
H = 128
R = 200
L = 2
SLOPE_NEG = (1.0 / 8.0 + 1.0 / 3.0) / 2.0

NC = 2
NS = 16
NW = NC * NS
LANES = 16
CHUNK = 112
CPW = 90
NCH = NW * CPW
EP = NCH * CHUNK
CPC = NCH // NC
VECS = CHUNK // LANES
ROWS_PER_TILE = 632
NP = NS * ROWS_PER_TILE
NR = N * R
CW_PER_TILE = NR // NS
CWO = 125056
CG = 8
NF = CPC // CG
CPW0 = 120
CPW1 = 60


def _gru_body(nei, wu, uu, bu, wr, ur, br, wh, uh, bh, w_out):
    q = nei[0]
    upd = jax.nn.sigmoid(jnp.dot(wu[0] + uu[0], q, preferred_element_type=jnp.float32) + bu[0])
    rst = jax.nn.sigmoid(jnp.dot(wr[0] + ur[0], q, preferred_element_type=jnp.float32) + br[0])
    hcap = jnp.tanh(
        jnp.dot(wh[0], q, preferred_element_type=jnp.float32)
        + jnp.dot(uh[0], rst * q, preferred_element_type=jnp.float32)
        + bh[0]
    )
    w_out[0] = (1.0 - upd) * q + upd * hcap


def _gru(nei_W, Wu, Uu, bu, Wr, Ur, br, Wh, Uh, bh):
    mat_spec = pl.BlockSpec((1, H, H), lambda i: (i, 0, 0))
    return pl.pallas_call(
        _gru_body,
        grid=(L,),
        in_specs=[mat_spec] * 10,
        out_specs=mat_spec,
        out_shape=jax.ShapeDtypeStruct((L, H, H), jnp.float32),
    )(nei_W, Wu, Uu, bu, Wr, Ur, br, Wh, Uh, bh)


def _sc_count_body(cidx_hbm, out_hbm, idx_v, cnt_v, sems):
    c = lax.axis_index("c")
    s = lax.axis_index("s")
    fbase = c * NF
    lo = s * CW_PER_TILE
    obase = (c * NS + s) * CWO
    (sem_i,) = sems
    ones16 = jnp.full((LANES,), 1.0, jnp.float32)

    def zstep(i, carry):
        cnt_v[pl.ds(i * LANES, LANES)] = jnp.zeros((LANES,), jnp.float32)
        return carry

    lax.fori_loop(0, CWO // LANES, zstep, 0)

    def start_idx(buf, f):
        pltpu.async_copy(cidx_hbm.at[pl.ds((fbase + f) * CG, CG)], idx_v.at[buf],
                         sem_i[buf])

    def wait_idx(buf, f):
        pltpu.make_async_copy(cidx_hbm.at[pl.ds((fbase + f) * CG, CG)],
                              idx_v.at[buf], sem_i[buf]).wait()

    start_idx(0, 0)
    start_idx(1, 1)
    start_idx(2, 2)

    def step(j, carry):
        for b in range(4):
            f = 4 * j + b

            @pl.when(f + 3 < NF)
            def _():
                start_idx((b + 3) % 4, f + 3)

            wait_idx(b, f)
            for q in range(CG):
                for g in range(VECS):
                    cidx = idx_v[b, q, pl.ds(g * LANES, LANES)]
                    local = cidx - lo
                    mask = (local >= 0) & (local < CW_PER_TILE)
                    local = jnp.where(mask, local, CW_PER_TILE)
                    plsc.addupdate_scatter(cnt_v, [local], ones16)
        return carry

    lax.fori_loop(0, NF // 4, step, 0)
    pltpu.sync_copy(cnt_v, out_hbm.at[pl.ds(obase, CWO)])


@functools.partial(
    pl.kernel,
    out_type=jax.ShapeDtypeStruct((NC * NS * CWO,), jnp.float32),
    mesh=plsc.VectorSubcoreMesh(core_axis_name="c", subcore_axis_name="s"),
    scratch_types=[
        pltpu.VMEM((4, CG, CHUNK), jnp.int32),
        pltpu.VMEM((CWO,), jnp.float32),
        [[pltpu.SemaphoreType.DMA] * 4],
    ],
    compiler_params=pltpu.CompilerParams(needs_layout_passes=False),
)
def _sc_count(cidx_hbm, out_hbm, idx_v, cnt_v, sems):
    _sc_count_body(cidx_hbm, out_hbm, idx_v, cnt_v, sems)


def _sc_accum_body(h_hbm, aux_hbm, zeros_hbm, out_hbm, idx_v, hrows, acc_sh, sems):
    c = lax.axis_index("c")
    s = lax.axis_index("s")
    base = jnp.where(c == 0, s * CPW0, NS * CPW0 + s * CPW1)
    cpw = jnp.where(c == 0, CPW0, CPW1)
    sem_i, sem_h, sem_s = sems

    def wait_scatter(buf):
        pltpu.make_async_copy(hrows.at[buf], acc_sh.at[idx_v.at[buf, 2]],
                              sem_s[buf]).wait()

    def prep(c2, buf):
        @pl.when(c2 < cpw)
        def _():
            @pl.when(c2 >= 3)
            def _():
                wait_scatter(buf)
            pltpu.async_copy(aux_hbm.at[base + c2], idx_v.at[buf], sem_i[buf])

    def gath(c1, buf):
        @pl.when(c1 < cpw)
        def _():
            pltpu.make_async_copy(aux_hbm.at[base + c1], idx_v.at[buf],
                                  sem_i[buf]).wait()
            pltpu.async_copy(h_hbm.at[idx_v.at[buf, 0]], hrows.at[buf], sem_h[buf])

    pltpu.sync_copy(zeros_hbm.at[pl.ds(s * ROWS_PER_TILE, ROWS_PER_TILE)],
                    acc_sh.at[pl.ds(s * ROWS_PER_TILE, ROWS_PER_TILE)])
    plsc.subcore_barrier()

    prep(0, 0)
    prep(1, 1)
    gath(0, 0)

    def step(j, carry):
        for b in range(3):
            ch = 3 * j + b
            prep(ch + 2, (b + 2) % 3)
            gath(ch + 1, (b + 1) % 3)
            pltpu.make_async_copy(h_hbm.at[idx_v.at[b, 0]], hrows.at[b],
                                  sem_h[b]).wait()
            pltpu.async_copy(hrows.at[b], acc_sh.at[idx_v.at[b, 2]], sem_s[b],
                             add=True)
        return carry

    lax.fori_loop(0, cpw // 3, step, 0)
    for buf in (0, 1, 2):
        wait_scatter(buf)
    plsc.subcore_barrier()
    pltpu.sync_copy(acc_sh.at[pl.ds(s * ROWS_PER_TILE, ROWS_PER_TILE)],
                    out_hbm.at[c, pl.ds(s * ROWS_PER_TILE, ROWS_PER_TILE)])


@functools.partial(
    pl.kernel,
    out_type=jax.ShapeDtypeStruct((NC, NP, H), jnp.float32),
    mesh=plsc.VectorSubcoreMesh(core_axis_name="c", subcore_axis_name="s"),
    scratch_types=[
        pltpu.VMEM((3, 4, CHUNK), jnp.int32),
        pltpu.VMEM((3, CHUNK, H), jnp.float32),
        pltpu.VMEM_SHARED((NP, H), jnp.float32),
        [[pltpu.SemaphoreType.DMA] * 3] * 3,
    ],
)
def _sc_accum(h_hbm, aux_hbm, zeros_hbm, out_hbm, idx_v, hrows, acc_sh, sems):
    _sc_accum_body(h_hbm, aux_hbm, zeros_hbm, out_hbm, idx_v, hrows, acc_sh, sems)


BN = 1000


def _combine_body(p_ref, c_ref, rel_ref, w_ref, o_ref):
    acc = p_ref[0] + p_ref[1]
    cnt = c_ref[0] + c_ref[1]
    acc = acc - jnp.dot(cnt, rel_ref[...], preferred_element_type=jnp.float32)
    o = jnp.dot(acc, w_ref[...], preferred_element_type=jnp.float32)
    o_ref[...] = jnp.where(o >= 0, o, o * SLOPE_NEG)


def _combine(p, cnt, rel, w):
    return pl.pallas_call(
        _combine_body,
        grid=(N // BN,),
        in_specs=[
            pl.BlockSpec((2, BN, H), lambda i: (0, i, 0)),
            pl.BlockSpec((2, BN, R), lambda i: (0, i, 0)),
            pl.BlockSpec((R, H), lambda i: (0, 0)),
            pl.BlockSpec((H, H), lambda i: (0, 0)),
        ],
        out_specs=pl.BlockSpec((BN, H), lambda i: (i, 0)),
        out_shape=jax.ShapeDtypeStruct((N, H), jnp.float32),
    )(p, cnt, rel, w)


def kernel(init_ent_emb, init_rel_emb, edge_index, edge_type, node_id,
           Wu, Uu, bu, Wr, Ur, br, Wh, Uh, bh, nei_W):
    h = jnp.take(init_ent_emb, node_id, axis=0)
    w_ev = _gru(nei_W, Wu, Uu, bu, Wr, Ur, br, Wh, Uh, bh)
    pad = EP - E
    src = jnp.concatenate([edge_index[0], jnp.zeros((pad,), jnp.int32)])
    dst = jnp.concatenate([edge_index[1], jnp.full((pad,), N, jnp.int32)])
    et = jnp.concatenate([edge_type, jnp.zeros((pad,), jnp.int32)])
    cidx = dst * R + et
    aux = jnp.stack([src.reshape(NCH, CHUNK), et.reshape(NCH, CHUNK),
                     dst.reshape(NCH, CHUNK), cidx.reshape(NCH, CHUNK)],
                    axis=1)
    zeros_a = jnp.zeros((NP, H), jnp.float32)
    cnt = _sc_count(cidx.reshape(NCH, CHUNK))
    cnt_m = cnt.reshape(NC, NS, CWO)[:, :, :CW_PER_TILE].reshape(NC, N, R)
    for l in range(L):
        p = _sc_accum(h, aux, zeros_a)
        h = _combine(p[:, :N], cnt_m, init_rel_emb[l], w_ev[l])
    return h


# speedup vs baseline: 2.4837x; 1.2476x over previous
import functools

import jax
import jax.numpy as jnp
from jax import lax
from jax.experimental import pallas as pl
from jax.experimental.pallas import tpu as pltpu
from jax.experimental.pallas import tpu_sc as plsc

N = 10000
E = 320000
H = 128
R = 200
L = 2
SLOPE_NEG = (1.0 / 8.0 + 1.0 / 3.0) / 2.0

NC = 2
NS = 16
NW = NC * NS
LANES = 16
CHUNK = 88
NCH = 3648
EP = NCH * CHUNK
ROWS_PER_TILE = 632
NP = NS * ROWS_PER_TILE
NR = N * R
CW_PER_TILE = NR // NS
CWO = 125056
CB = 1408
CWC = EP // NC
NF = CWC // CB
CVE = CB // LANES
CPW0 = 152
CPW1 = 76


def _gru_body(nei, wu, uu, bu, wr, ur, br, wh, uh, bh, w_out):
    q = nei[0]
    upd = jax.nn.sigmoid(jnp.dot(wu[0] + uu[0], q, preferred_element_type=jnp.float32) + bu[0])
    rst = jax.nn.sigmoid(jnp.dot(wr[0] + ur[0], q, preferred_element_type=jnp.float32) + br[0])
    hcap = jnp.tanh(
        jnp.dot(wh[0], q, preferred_element_type=jnp.float32)
        + jnp.dot(uh[0], rst * q, preferred_element_type=jnp.float32)
        + bh[0]
    )
    w_out[0] = (1.0 - upd) * q + upd * hcap


def _gru(nei_W, Wu, Uu, bu, Wr, Ur, br, Wh, Uh, bh):
    mat_spec = pl.BlockSpec((1, H, H), lambda i: (i, 0, 0))
    return pl.pallas_call(
        _gru_body,
        grid=(L,),
        in_specs=[mat_spec] * 10,
        out_specs=mat_spec,
        out_shape=jax.ShapeDtypeStruct((L, H, H), jnp.float32),
    )(nei_W, Wu, Uu, bu, Wr, Ur, br, Wh, Uh, bh)


def _sc_count_body(cidx_hbm, out_hbm, idx_v, cnt_v, sems):
    c = lax.axis_index("c")
    s = lax.axis_index("s")
    fbase = c * CWC
    lo = s * CW_PER_TILE
    obase = (c * NS + s) * CWO
    (sem_i,) = sems
    ones16 = jnp.full((LANES,), 1.0, jnp.float32)

    def zstep(i, carry):
        cnt_v[pl.ds(i * LANES, LANES)] = jnp.zeros((LANES,), jnp.float32)
        return carry

    lax.fori_loop(0, CWO // LANES, zstep, 0)

    def start_idx(buf, f):
        pltpu.async_copy(cidx_hbm.at[pl.ds(fbase + f * CB, CB)], idx_v.at[buf],
                         sem_i[buf])

    def wait_idx(buf, f):
        pltpu.make_async_copy(cidx_hbm.at[pl.ds(fbase + f * CB, CB)],
                              idx_v.at[buf], sem_i[buf]).wait()

    start_idx(0, 0)

    def step(j, carry):
        for b in range(2):
            f = 2 * j + b

            @pl.when(f + 1 < NF)
            def _():
                start_idx(1 - b, f + 1)

            wait_idx(b, f)
            for g in range(CVE):
                cidx = idx_v[b, pl.ds(g * LANES, LANES)]
                local = cidx - lo
                mask = (local >= 0) & (local < CW_PER_TILE)
                local = jnp.where(mask, local, CW_PER_TILE)
                plsc.addupdate_scatter(cnt_v, [local], ones16)
        return carry

    lax.fori_loop(0, NF // 2, step, 0)
    pltpu.sync_copy(cnt_v, out_hbm.at[pl.ds(obase, CWO)])


@functools.partial(
    pl.kernel,
    out_type=jax.ShapeDtypeStruct((NC * NS * CWO,), jnp.float32),
    mesh=plsc.VectorSubcoreMesh(core_axis_name="c", subcore_axis_name="s"),
    scratch_types=[
        pltpu.VMEM((2, CB), jnp.int32),
        pltpu.VMEM((CWO,), jnp.float32),
        [[pltpu.SemaphoreType.DMA] * 2],
    ],
    compiler_params=pltpu.CompilerParams(needs_layout_passes=False),
)
def _sc_count(cidx_hbm, out_hbm, idx_v, cnt_v, sems):
    _sc_count_body(cidx_hbm, out_hbm, idx_v, cnt_v, sems)


def _sc_accum_body(h_hbm, aux_hbm, zeros_hbm, out_hbm, idx_v, hrows, acc_sh, sems):
    c = lax.axis_index("c")
    s = lax.axis_index("s")
    base = jnp.where(c == 0, s * CPW0, NS * CPW0 + s * CPW1)
    cpw = jnp.where(c == 0, CPW0, CPW1)
    sem_i, sem_h, sem_s = sems

    def wait_scatter(buf):
        pltpu.make_async_copy(hrows.at[buf], acc_sh.at[idx_v.at[buf, 1]],
                              sem_s[buf]).wait()

    def prep(c2, buf):
        @pl.when(c2 < cpw)
        def _():
            @pl.when(c2 >= 4)
            def _():
                wait_scatter(buf)
            pltpu.async_copy(aux_hbm.at[base + c2], idx_v.at[buf], sem_i[buf])

    def gath(c1, buf):
        @pl.when(c1 < cpw)
        def _():
            pltpu.make_async_copy(aux_hbm.at[base + c1], idx_v.at[buf],
                                  sem_i[buf]).wait()
            pltpu.async_copy(h_hbm.at[idx_v.at[buf, 0]], hrows.at[buf], sem_h[buf])

    pltpu.sync_copy(zeros_hbm.at[pl.ds(s * ROWS_PER_TILE, ROWS_PER_TILE)],
                    acc_sh.at[pl.ds(s * ROWS_PER_TILE, ROWS_PER_TILE)])
    plsc.subcore_barrier()

    prep(0, 0)
    prep(1, 1)
    prep(2, 2)
    gath(0, 0)
    gath(1, 1)

    def step(j, carry):
        for b in range(4):
            ch = 4 * j + b
            prep(ch + 3, (b + 3) % 4)
            gath(ch + 2, (b + 2) % 4)
            pltpu.make_async_copy(h_hbm.at[idx_v.at[b, 0]], hrows.at[b],
                                  sem_h[b]).wait()
            pltpu.async_copy(hrows.at[b], acc_sh.at[idx_v.at[b, 1]], sem_s[b],
                             add=True)
        return carry

    lax.fori_loop(0, cpw // 4, step, 0)
    for buf in (0, 1, 2, 3):
        wait_scatter(buf)
    plsc.subcore_barrier()
    pltpu.sync_copy(acc_sh.at[pl.ds(s * ROWS_PER_TILE, ROWS_PER_TILE)],
                    out_hbm.at[c, pl.ds(s * ROWS_PER_TILE, ROWS_PER_TILE)])


@functools.partial(
    pl.kernel,
    out_type=jax.ShapeDtypeStruct((NC, NP, H), jnp.float32),
    mesh=plsc.VectorSubcoreMesh(core_axis_name="c", subcore_axis_name="s"),
    scratch_types=[
        pltpu.VMEM((4, 2, CHUNK), jnp.int32),
        pltpu.VMEM((4, CHUNK, H), jnp.float32),
        pltpu.VMEM_SHARED((NP, H), jnp.float32),
        [[pltpu.SemaphoreType.DMA] * 4] * 3,
    ],
)
def _sc_accum(h_hbm, aux_hbm, zeros_hbm, out_hbm, idx_v, hrows, acc_sh, sems):
    _sc_accum_body(h_hbm, aux_hbm, zeros_hbm, out_hbm, idx_v, hrows, acc_sh, sems)


BN = 1000


def _combine_body(p_ref, c_ref, rel_ref, w_ref, o_ref):
    acc = p_ref[0] + p_ref[1]
    cnt = c_ref[0] + c_ref[1]
    acc = acc - jnp.dot(cnt, rel_ref[...], preferred_element_type=jnp.float32)
    o = jnp.dot(acc, w_ref[...], preferred_element_type=jnp.float32)
    o_ref[...] = jnp.where(o >= 0, o, o * SLOPE_NEG)


def _combine(p, cnt, rel, w):
    return pl.pallas_call(
        _combine_body,
        grid=(N // BN,),
        in_specs=[
            pl.BlockSpec((2, BN, H), lambda i: (0, i, 0)),
            pl.BlockSpec((2, BN, R), lambda i: (0, i, 0)),
            pl.BlockSpec((R, H), lambda i: (0, 0)),
            pl.BlockSpec((H, H), lambda i: (0, 0)),
        ],
        out_specs=pl.BlockSpec((BN, H), lambda i: (i, 0)),
        out_shape=jax.ShapeDtypeStruct((N, H), jnp.float32),
    )(p, cnt, rel, w)


def kernel(init_ent_emb, init_rel_emb, edge_index, edge_type, node_id,
           Wu, Uu, bu, Wr, Ur, br, Wh, Uh, bh, nei_W):
    h = jnp.take(init_ent_emb, node_id, axis=0)
    w_ev = _gru(nei_W, Wu, Uu, bu, Wr, Ur, br, Wh, Uh, bh)
    pad = EP - E
    src = jnp.concatenate([edge_index[0], jnp.zeros((pad,), jnp.int32)])
    dst = jnp.concatenate([edge_index[1], jnp.full((pad,), N, jnp.int32)])
    et = jnp.concatenate([edge_type, jnp.zeros((pad,), jnp.int32)])
    cidx = dst * R + et
    aux = jnp.stack([src.reshape(NCH, CHUNK), dst.reshape(NCH, CHUNK)],
                    axis=1)
    zeros_a = jnp.zeros((NP, H), jnp.float32)
    cnt = _sc_count(cidx)
    cnt_m = cnt.reshape(NC, NS, CWO)[:, :, :CW_PER_TILE].reshape(NC, N, R)
    for l in range(L):
        p = _sc_accum(h, aux, zeros_a)
        h = _combine(p[:, :N], cnt_m, init_rel_emb[l], w_ev[l])
    return h


# speedup vs baseline: 2.6680x; 1.0742x over previous
import functools

import jax
import jax.numpy as jnp
from jax import lax
from jax.experimental import pallas as pl
from jax.experimental.pallas import tpu as pltpu
from jax.experimental.pallas import tpu_sc as plsc

N = 10000
E = 320000
H = 128
R = 200
L = 2
SLOPE_NEG = (1.0 / 8.0 + 1.0 / 3.0) / 2.0

NC = 2
NS = 16
NW = NC * NS
LANES = 16
CHUNK = 88
NCH = 3648
EP = NCH * CHUNK
ROWS_PER_TILE = 632
NP = NS * ROWS_PER_TILE
NR = N * R
CW_PER_TILE = NR // NS
CWO = 125056
CB = 1408
CWC = EP // NC
NF = CWC // CB
CVE = CB // LANES
CPW0 = 152
CPW1 = 76


def _gru_body(nei, wu, uu, bu, wr, ur, br, wh, uh, bh, w_out):
    q = nei[0]
    upd = jax.nn.sigmoid(jnp.dot(wu[0] + uu[0], q, preferred_element_type=jnp.float32) + bu[0])
    rst = jax.nn.sigmoid(jnp.dot(wr[0] + ur[0], q, preferred_element_type=jnp.float32) + br[0])
    hcap = jnp.tanh(
        jnp.dot(wh[0], q, preferred_element_type=jnp.float32)
        + jnp.dot(uh[0], rst * q, preferred_element_type=jnp.float32)
        + bh[0]
    )
    w_out[0] = (1.0 - upd) * q + upd * hcap


def _gru(nei_W, Wu, Uu, bu, Wr, Ur, br, Wh, Uh, bh):
    mat_spec = pl.BlockSpec((1, H, H), lambda i: (i, 0, 0))
    return pl.pallas_call(
        _gru_body,
        grid=(L,),
        in_specs=[mat_spec] * 10,
        out_specs=mat_spec,
        out_shape=jax.ShapeDtypeStruct((L, H, H), jnp.float32),
    )(nei_W, Wu, Uu, bu, Wr, Ur, br, Wh, Uh, bh)


def _sc_count_body(cidx_hbm, out_hbm, idx_v, cnt_v, sems):
    c = lax.axis_index("c")
    s = lax.axis_index("s")
    fbase = c * CWC
    lo = s * CW_PER_TILE
    obase = (c * NS + s) * CWO
    (sem_i,) = sems
    ones16 = jnp.full((LANES,), 1.0, jnp.float32)

    def zstep(i, carry):
        cnt_v[pl.ds(i * LANES, LANES)] = jnp.zeros((LANES,), jnp.float32)
        return carry

    lax.fori_loop(0, CWO // LANES, zstep, 0)

    def start_idx(buf, f):
        pltpu.async_copy(cidx_hbm.at[pl.ds(fbase + f * CB, CB)], idx_v.at[buf],
                         sem_i[buf])

    def wait_idx(buf, f):
        pltpu.make_async_copy(cidx_hbm.at[pl.ds(fbase + f * CB, CB)],
                              idx_v.at[buf], sem_i[buf]).wait()

    start_idx(0, 0)

    def step(j, carry):
        for b in range(2):
            f = 2 * j + b

            @pl.when(f + 1 < NF)
            def _():
                start_idx(1 - b, f + 1)

            wait_idx(b, f)

            @plsc.parallel_loop(0, CB, step=LANES, unroll=8)
            def _(g):
                cidx = idx_v[b, pl.ds(g, LANES)]
                local = cidx - lo
                mask = (local >= 0) & (local < CW_PER_TILE)
                local = jnp.where(mask, local, CW_PER_TILE)
                plsc.addupdate_scatter(cnt_v, [local], ones16)
        return carry

    lax.fori_loop(0, NF // 2, step, 0)
    pltpu.sync_copy(cnt_v, out_hbm.at[pl.ds(obase, CWO)])


@functools.partial(
    pl.kernel,
    out_type=jax.ShapeDtypeStruct((NC * NS * CWO,), jnp.float32),
    mesh=plsc.VectorSubcoreMesh(core_axis_name="c", subcore_axis_name="s"),
    scratch_types=[
        pltpu.VMEM((2, CB), jnp.int32),
        pltpu.VMEM((CWO,), jnp.float32),
        [[pltpu.SemaphoreType.DMA] * 2],
    ],
    compiler_params=pltpu.CompilerParams(needs_layout_passes=False),
)
def _sc_count(cidx_hbm, out_hbm, idx_v, cnt_v, sems):
    _sc_count_body(cidx_hbm, out_hbm, idx_v, cnt_v, sems)


def _sc_accum_body(h_hbm, aux_hbm, zeros_hbm, out_hbm, idx_v, hrows, acc_sh, sems):
    c = lax.axis_index("c")
    s = lax.axis_index("s")
    base = jnp.where(c == 0, s * CPW0, NS * CPW0 + s * CPW1)
    cpw = jnp.where(c == 0, CPW0, CPW1)
    sem_i, sem_h, sem_s = sems

    def wait_scatter(buf):
        pltpu.make_async_copy(hrows.at[buf], acc_sh.at[idx_v.at[buf, 1]],
                              sem_s[buf]).wait()

    def prep(c2, buf):
        @pl.when(c2 < cpw)
        def _():
            @pl.when(c2 >= 4)
            def _():
                wait_scatter(buf)
            pltpu.async_copy(aux_hbm.at[base + c2], idx_v.at[buf], sem_i[buf])

    def gath(c1, buf):
        @pl.when(c1 < cpw)
        def _():
            pltpu.make_async_copy(aux_hbm.at[base + c1], idx_v.at[buf],
                                  sem_i[buf]).wait()
            pltpu.async_copy(h_hbm.at[idx_v.at[buf, 0]], hrows.at[buf], sem_h[buf])

    pltpu.sync_copy(zeros_hbm.at[pl.ds(s * ROWS_PER_TILE, ROWS_PER_TILE)],
                    acc_sh.at[pl.ds(s * ROWS_PER_TILE, ROWS_PER_TILE)])
    plsc.subcore_barrier()

    prep(0, 0)
    prep(1, 1)
    prep(2, 2)
    gath(0, 0)
    gath(1, 1)

    def step(j, carry):
        for b in range(4):
            ch = 4 * j + b
            prep(ch + 3, (b + 3) % 4)
            gath(ch + 2, (b + 2) % 4)
            pltpu.make_async_copy(h_hbm.at[idx_v.at[b, 0]], hrows.at[b],
                                  sem_h[b]).wait()
            pltpu.async_copy(hrows.at[b], acc_sh.at[idx_v.at[b, 1]], sem_s[b],
                             add=True)
        return carry

    lax.fori_loop(0, cpw // 4, step, 0)
    for buf in (0, 1, 2, 3):
        wait_scatter(buf)
    plsc.subcore_barrier()
    pltpu.sync_copy(acc_sh.at[pl.ds(s * ROWS_PER_TILE, ROWS_PER_TILE)],
                    out_hbm.at[c, pl.ds(s * ROWS_PER_TILE, ROWS_PER_TILE)])


@functools.partial(
    pl.kernel,
    out_type=jax.ShapeDtypeStruct((NC, NP, H), jnp.float32),
    mesh=plsc.VectorSubcoreMesh(core_axis_name="c", subcore_axis_name="s"),
    scratch_types=[
        pltpu.VMEM((4, 2, CHUNK), jnp.int32),
        pltpu.VMEM((4, CHUNK, H), jnp.float32),
        pltpu.VMEM_SHARED((NP, H), jnp.float32),
        [[pltpu.SemaphoreType.DMA] * 4] * 3,
    ],
)
def _sc_accum(h_hbm, aux_hbm, zeros_hbm, out_hbm, idx_v, hrows, acc_sh, sems):
    _sc_accum_body(h_hbm, aux_hbm, zeros_hbm, out_hbm, idx_v, hrows, acc_sh, sems)


BN = 1000


def _combine_body(p_ref, c_ref, rel_ref, w_ref, o_ref):
    acc = p_ref[0] + p_ref[1]
    cnt = c_ref[0] + c_ref[1]
    acc = acc - jnp.dot(cnt, rel_ref[...], preferred_element_type=jnp.float32)
    o = jnp.dot(acc, w_ref[...], preferred_element_type=jnp.float32)
    o_ref[...] = jnp.where(o >= 0, o, o * SLOPE_NEG)


def _combine(p, cnt, rel, w):
    return pl.pallas_call(
        _combine_body,
        grid=(N // BN,),
        in_specs=[
            pl.BlockSpec((2, BN, H), lambda i: (0, i, 0)),
            pl.BlockSpec((2, BN, R), lambda i: (0, i, 0)),
            pl.BlockSpec((R, H), lambda i: (0, 0)),
            pl.BlockSpec((H, H), lambda i: (0, 0)),
        ],
        out_specs=pl.BlockSpec((BN, H), lambda i: (i, 0)),
        out_shape=jax.ShapeDtypeStruct((N, H), jnp.float32),
    )(p, cnt, rel, w)


def kernel(init_ent_emb, init_rel_emb, edge_index, edge_type, node_id,
           Wu, Uu, bu, Wr, Ur, br, Wh, Uh, bh, nei_W):
    h = jnp.take(init_ent_emb, node_id, axis=0)
    w_ev = _gru(nei_W, Wu, Uu, bu, Wr, Ur, br, Wh, Uh, bh)
    pad = EP - E
    src = jnp.concatenate([edge_index[0], jnp.zeros((pad,), jnp.int32)])
    dst = jnp.concatenate([edge_index[1], jnp.full((pad,), N, jnp.int32)])
    et = jnp.concatenate([edge_type, jnp.zeros((pad,), jnp.int32)])
    cidx = dst * R + et
    aux = jnp.stack([src.reshape(NCH, CHUNK), dst.reshape(NCH, CHUNK)],
                    axis=1)
    zeros_a = jnp.zeros((NP, H), jnp.float32)
    cnt = _sc_count(cidx)
    cnt_m = cnt.reshape(NC, NS, CWO)[:, :, :CW_PER_TILE].reshape(NC, N, R)
    for l in range(L):
        p = _sc_accum(h, aux, zeros_a)
        h = _combine(p[:, :N], cnt_m, init_rel_emb[l], w_ev[l])
    return h
